# Initial kernel scaffold; baseline (speedup 1.0000x reference)
import jax
import jax.numpy as jnp
from jax.experimental import pallas as pl


def kernel(input, embedding_weight):
    raise NotImplementedError("write your pallas kernel here")


# speedup vs baseline: 3.3021x; 3.3021x over previous
import functools

import jax
import jax.numpy as jnp
from jax import lax
from jax.experimental import pallas as pl
from jax.experimental.pallas import tpu as pltpu
from jax.experimental.pallas import tpu_sc as plsc

HIDDEN = 128
NUM_CORES = 2
NUM_SUBCORES = 16
NW = NUM_CORES * NUM_SUBCORES
CHUNK = 800


@functools.lru_cache(maxsize=None)
def _make_gather(n_idx: int):
    assert n_idx % (8 * NW) == 0
    b_per_w = n_idx // NW
    assert b_per_w % CHUNK == 0
    n_chunks = b_per_w // CHUNK
    mesh = plsc.VectorSubcoreMesh(core_axis_name="c", subcore_axis_name="s")

    @functools.partial(
        pl.kernel,
        mesh=mesh,
        out_type=jax.ShapeDtypeStruct((n_idx, HIDDEN), jnp.float32),
        scratch_types=[
            pltpu.VMEM((CHUNK,), jnp.int32),
            pltpu.VMEM((CHUNK, HIDDEN), jnp.float32),
            pltpu.SemaphoreType.DMA,
        ],
    )
    def gather_kernel(table_hbm, idx_hbm, out_hbm, idx_v, rows_v, sem):
        wid = lax.axis_index("s") * NUM_CORES + lax.axis_index("c")
        base = wid * b_per_w
        for c in range(n_chunks):
            off = base + c * CHUNK
            pltpu.sync_copy(idx_hbm.at[pl.ds(off, CHUNK)], idx_v)
            pltpu.async_copy(table_hbm.at[idx_v], rows_v, sem).wait()
            pltpu.sync_copy(rows_v, out_hbm.at[pl.ds(off, CHUNK)])

    return gather_kernel


@jax.jit
def kernel(input, embedding_weight):
    b, s = input.shape
    idx = input.reshape(b * s).astype(jnp.int32)
    out = _make_gather(b * s)(embedding_weight, idx)
    return out.reshape(b, s, HIDDEN)


# speedup vs baseline: 3.3282x; 1.0079x over previous
import functools

import jax
import jax.numpy as jnp
from jax import lax
from jax.experimental import pallas as pl
from jax.experimental.pallas import tpu as pltpu
from jax.experimental.pallas import tpu_sc as plsc

HIDDEN = 128
NUM_CORES = 2
NUM_SUBCORES = 16
NW = NUM_CORES * NUM_SUBCORES
CHUNK = 400


@functools.lru_cache(maxsize=None)
def _make_gather(n_idx: int):
    assert n_idx % (NW * CHUNK) == 0
    b_per_w = n_idx // NW
    n_chunks = b_per_w // CHUNK
    mesh = plsc.VectorSubcoreMesh(core_axis_name="c", subcore_axis_name="s")

    @functools.partial(
        pl.kernel,
        mesh=mesh,
        out_type=jax.ShapeDtypeStruct((n_idx, HIDDEN), jnp.float32),
        scratch_types=[
            pltpu.VMEM((CHUNK,), jnp.int32),
            pltpu.VMEM((CHUNK,), jnp.int32),
            pltpu.VMEM((CHUNK, HIDDEN), jnp.float32),
            pltpu.VMEM((CHUNK, HIDDEN), jnp.float32),
            pltpu.SemaphoreType.DMA,
            pltpu.SemaphoreType.DMA,
            pltpu.SemaphoreType.DMA,
            pltpu.SemaphoreType.DMA,
        ],
    )
    def gather_kernel(table_hbm, idx_hbm, out_hbm, idx_v0, idx_v1,
                      rows_v0, rows_v1, gsem0, gsem1, wsem0, wsem1):
        wid = lax.axis_index("s") * NUM_CORES + lax.axis_index("c")
        base = wid * b_per_w
        idx = (idx_v0, idx_v1)
        rows = (rows_v0, rows_v1)
        gsem = (gsem0, gsem1)
        wsem = (wsem0, wsem1)

        gathers = [None, None]
        writes = [None, None]
        pltpu.sync_copy(idx_hbm.at[pl.ds(base, CHUNK)], idx[0])
        gathers[0] = pltpu.async_copy(table_hbm.at[idx[0]], rows[0], gsem[0])
        for c in range(n_chunks):
            b = c % 2
            nb = (c + 1) % 2
            if c + 1 < n_chunks:
                if writes[nb] is not None:
                    writes[nb].wait()
                pltpu.sync_copy(
                    idx_hbm.at[pl.ds(base + (c + 1) * CHUNK, CHUNK)], idx[nb])
                gathers[nb] = pltpu.async_copy(
                    table_hbm.at[idx[nb]], rows[nb], gsem[nb])
            gathers[b].wait()
            writes[b] = pltpu.async_copy(
                rows[b], out_hbm.at[pl.ds(base + c * CHUNK, CHUNK)], wsem[b])
        for w in writes:
            w.wait()

    return gather_kernel


@jax.jit
def kernel(input, embedding_weight):
    b, s = input.shape
    n_idx = b * s
    idx = input.reshape(n_idx).astype(jnp.int32)
    out = _make_gather(n_idx)(embedding_weight, idx)
    return out.reshape(b, s, HIDDEN)


# speedup vs baseline: 5.8042x; 1.7440x over previous
import functools

import jax
import jax.numpy as jnp
from jax import lax
from jax.experimental import pallas as pl
from jax.experimental.pallas import tpu as pltpu
from jax.experimental.pallas import tpu_sc as plsc

HIDDEN = 128
NUM_CORES = 2
NUM_SUBCORES = 16
NW = NUM_CORES * NUM_SUBCORES
CHUNK_B = 8


@functools.lru_cache(maxsize=None)
def _make_gather(batch: int, seq: int):
    assert batch % (NW * CHUNK_B) == 0
    b_per_w = batch // NW
    n_chunks = b_per_w // CHUNK_B
    chunk = CHUNK_B * seq
    mesh = plsc.VectorSubcoreMesh(core_axis_name="c", subcore_axis_name="s")

    @functools.partial(
        pl.kernel,
        mesh=mesh,
        out_type=jax.ShapeDtypeStruct((batch, seq, HIDDEN), jnp.float32),
        scratch_types=[
            pltpu.VMEM((chunk,), jnp.int32),
            pltpu.VMEM((chunk,), jnp.int32),
            pltpu.VMEM((chunk, HIDDEN), jnp.float32),
            pltpu.VMEM((chunk, HIDDEN), jnp.float32),
            pltpu.SemaphoreType.DMA,
            pltpu.SemaphoreType.DMA,
            pltpu.SemaphoreType.DMA,
            pltpu.SemaphoreType.DMA,
        ],
    )
    def gather_kernel(table_hbm, idx_hbm, out_hbm, idx_v0, idx_v1,
                      rows_v0, rows_v1, gsem0, gsem1, wsem0, wsem1):
        wid = lax.axis_index("s") * NUM_CORES + lax.axis_index("c")
        bbase = wid * b_per_w
        idx = (idx_v0, idx_v1)
        rows = (rows_v0, rows_v1)
        gsem = (gsem0, gsem1)
        wsem = (wsem0, wsem1)

        gathers = [None, None]
        writes = [[], []]
        pltpu.sync_copy(idx_hbm.at[pl.ds(bbase * seq, chunk)], idx[0])
        gathers[0] = pltpu.async_copy(table_hbm.at[idx[0]], rows[0], gsem[0])
        for c in range(n_chunks):
            b = c % 2
            nb = (c + 1) % 2
            if c + 1 < n_chunks:
                for w in writes[nb]:
                    w.wait()
                off = (bbase + (c + 1) * CHUNK_B) * seq
                pltpu.sync_copy(idx_hbm.at[pl.ds(off, chunk)], idx[nb])
                gathers[nb] = pltpu.async_copy(
                    table_hbm.at[idx[nb]], rows[nb], gsem[nb])
            gathers[b].wait()
            writes[b] = [
                pltpu.async_copy(
                    rows[b].at[pl.ds(r * seq, seq)],
                    out_hbm.at[bbase + c * CHUNK_B + r],
                    wsem[b])
                for r in range(CHUNK_B)
            ]
        for ws in writes:
            for w in ws:
                w.wait()

    return gather_kernel


@jax.jit
def kernel(input, embedding_weight):
    b, s = input.shape
    idx = input.reshape(b * s).astype(jnp.int32)
    return _make_gather(b, s)(embedding_weight, idx)
